# baseline (speedup 1.0000x reference)
import functools

import jax
import jax.numpy as jnp
from jax import lax
from jax.experimental import pallas as pl
from jax.experimental.pallas import tpu as pltpu
from jax.experimental.pallas import tpu_sc as plsc

N = 10000
E = 320000
NT = 4
D_IN = 128
D_HID = 128
D_OUT = 64

NC = 2
NS = 16
CH = 128
G = 4
NCHUNK = 160
NG = NCHUNK // G
E_PT = NCHUNK * CH
EP = E_PT * NS
N_ACC = 10240
ROWS_PT = 640
PAD_DST = 10016

_mesh = plsc.VectorSubcoreMesh(core_axis_name="c", subcore_axis_name="s")
_sc_params = pltpu.CompilerParams(use_tc_tiling_on_sc=False)


def _lrelu(x):
    return jnp.where(x >= 0, x, 0.2 * x)


@functools.partial(
    pl.kernel,
    out_type=jax.ShapeDtypeStruct((NT, N_ACC), jnp.float32),
    mesh=_mesh,
    compiler_params=_sc_params,
    scratch_types=[
        pltpu.VMEM((2, CH), jnp.int32),
        pltpu.VMEM((2, CH), jnp.int32),
        pltpu.VMEM((2, CH), jnp.int32),
        pltpu.VMEM((2, CH), jnp.int32),
        pltpu.VMEM((CH,), jnp.float32),
        pltpu.VMEM((ROWS_PT,), jnp.float32),
        pltpu.VMEM_SHARED((N_ACC,), jnp.float32),
        pltpu.SemaphoreType.DMA,
        pltpu.SemaphoreType.DMA,
        pltpu.SemaphoreType.DMA,
        pltpu.SemaphoreType.DMA,
        pltpu.SemaphoreType.DMA,
        pltpu.SemaphoreType.DMA,
    ],
)
def _deg_kernel(idx_hbm, deg_out, ib0, ib1, ib2, ib3, ones_v, zeros_v, acc,
                isem0, isem1, isem2, isem3, ssem0, ssem1):
    c = lax.axis_index("c")
    s = lax.axis_index("s")
    ib = (ib0, ib1, ib2, ib3)
    isem = (isem0, isem1, isem2, isem3)
    ssem = (ssem0, ssem1)
    for i in range(8):
        ones_v[pl.ds(16 * i, 16)] = jnp.ones((16,), jnp.float32)
    for i in range(ROWS_PT // 16):
        zeros_v[pl.ds(16 * i, 16)] = jnp.zeros((16,), jnp.float32)
    for k in range(2):
        t = 2 * c + k
        pltpu.sync_copy(zeros_v, acc.at[pl.ds(s * ROWS_PT, ROWS_PT)])
        plsc.subcore_barrier()

        pltpu.sync_copy(idx_hbm.at[t, s, 0], ib0)
        pltpu.async_copy(idx_hbm.at[t, s, 1], ib1, isem1)

        def quad(q, _):
            for u in range(4):
                j = 4 * q + u
                I0, I2 = u % 4, (u + 2) % 4
                X = u % 2
                Y = 1 - X
                if u == 0:
                    @pl.when(j > 0)
                    def _():
                        pltpu.make_async_copy(ones_v, acc.at[ib[I0].at[1]],
                                              ssem[Y]).wait()
                else:
                    pltpu.make_async_copy(ones_v, acc.at[ib[I0].at[1]],
                                          ssem[Y]).wait()
                jp2 = jnp.minimum(j + 2, NCHUNK - 1)
                pltpu.async_copy(idx_hbm.at[t, s, jp2], ib[I2],
                                 isem[I2])
                if u == 0:
                    @pl.when(j > 0)
                    def _():
                        pltpu.make_async_copy(idx_hbm.at[t, s, 0],
                                              ib[I0], isem[I0]).wait()
                else:
                    pltpu.make_async_copy(idx_hbm.at[t, s, 0],
                                          ib[I0], isem[I0]).wait()
                pltpu.async_copy(ones_v, acc.at[ib[I0].at[1]], ssem[X],
                                 add=True)
            return 0
        lax.fori_loop(0, NCHUNK // 4, quad, 0)
        pltpu.make_async_copy(ones_v, acc.at[ib3.at[1]], ssem1).wait()
        pltpu.make_async_copy(idx_hbm.at[t, s, 0], ib0, isem0).wait()
        pltpu.make_async_copy(idx_hbm.at[t, s, 0], ib1, isem1).wait()
        plsc.subcore_barrier()

        pltpu.sync_copy(acc.at[pl.ds(s * ROWS_PT, ROWS_PT)],
                        deg_out.at[t, pl.ds(s * ROWS_PT, ROWS_PT)])


def _make_scatter_kernel(D):
    @functools.partial(
        pl.kernel,
        out_type=jax.ShapeDtypeStruct((NT, N, D), jnp.float32),
        mesh=_mesh,
        compiler_params=_sc_params,
        scratch_types=[
            pltpu.VMEM((2, CH), jnp.int32),
            pltpu.VMEM((2, CH), jnp.int32),
            pltpu.VMEM((2, CH), jnp.int32),
            pltpu.VMEM((2, CH), jnp.int32),
            pltpu.VMEM((CH, D), jnp.float32),
            pltpu.VMEM((CH, D), jnp.float32),
            pltpu.VMEM((64, D), jnp.float32),
            pltpu.VMEM_SHARED((N_ACC, D), jnp.float32),
            pltpu.SemaphoreType.DMA,
            pltpu.SemaphoreType.DMA,
            pltpu.SemaphoreType.DMA,
            pltpu.SemaphoreType.DMA,
            pltpu.SemaphoreType.DMA,
            pltpu.SemaphoreType.DMA,
            pltpu.SemaphoreType.DMA,
            pltpu.SemaphoreType.DMA,
        ],
    )
    def _scatter(idx_hbm, h_hbm, tmp_out, ib0, ib1, ib2, ib3, rows0, rows1,
                 zbig, acc, isem0, isem1, isem2, isem3,
                 gsem0, gsem1, ssem0, ssem1):
        c = lax.axis_index("c")
        s = lax.axis_index("s")
        ib = (ib0, ib1, ib2, ib3)
        isem = (isem0, isem1, isem2, isem3)
        rowsb = (rows0, rows1)
        gsem = (gsem0, gsem1)
        ssem = (ssem0, ssem1)
        def zfill(r, _):
            for q in range(D // 16):
                zbig[r, pl.ds(16 * q, 16)] = jnp.zeros((16,), jnp.float32)
            return 0
        lax.fori_loop(0, 64, zfill, 0)
        for k in range(2):
            t = 2 * c + k

            def zrow(j, _):
                pltpu.sync_copy(zbig, acc.at[pl.ds(s * ROWS_PT + 64 * j, 64)])
                return 0
            lax.fori_loop(0, ROWS_PT // 64, zrow, 0)
            plsc.subcore_barrier()

            pltpu.sync_copy(idx_hbm.at[t, s, 0], ib0)
            pltpu.async_copy(idx_hbm.at[t, s, 1], ib1, isem1)
            pltpu.async_copy(h_hbm.at[ib0.at[0]], rows0, gsem0)

            def quad(q, _):
                for u in range(4):
                    j = 4 * q + u
                    I0, I1, I2 = u % 4, (u + 1) % 4, (u + 2) % 4
                    X = u % 2
                    Y = 1 - X
                    if u == 0:
                        @pl.when(j > 0)
                        def _():
                            pltpu.make_async_copy(
                                rowsb[Y], acc.at[ib[I1].at[1]], ssem[Y]).wait()
                    else:
                        pltpu.make_async_copy(
                            rowsb[Y], acc.at[ib[I1].at[1]], ssem[Y]).wait()
                    jp2 = jnp.minimum(j + 2, NCHUNK - 1)
                    pltpu.async_copy(idx_hbm.at[t, s, jp2], ib[I2],
                                     isem[I2])
                    pltpu.make_async_copy(idx_hbm.at[t, s, 0], ib[I1],
                                          isem[I1]).wait()
                    pltpu.async_copy(h_hbm.at[ib[I1].at[0]], rowsb[Y],
                                     gsem[Y])
                    pltpu.make_async_copy(h_hbm.at[ib[I0].at[0]], rowsb[X],
                                          gsem[X]).wait()
                    pltpu.async_copy(rowsb[X], acc.at[ib[I0].at[1]],
                                     ssem[X], add=True)
                return 0
            lax.fori_loop(0, NCHUNK // 4, quad, 0)
            pltpu.make_async_copy(h_hbm.at[ib0.at[0]], rows0, gsem0).wait()
            pltpu.make_async_copy(rows1, acc.at[ib3.at[1]], ssem1).wait()
            pltpu.make_async_copy(idx_hbm.at[t, s, 0], ib1,
                                  isem1).wait()
            plsc.subcore_barrier()

            @pl.when(s < 15)
            def _():
                pltpu.sync_copy(acc.at[pl.ds(s * ROWS_PT, ROWS_PT)],
                                tmp_out.at[t, pl.ds(s * ROWS_PT, ROWS_PT)])

            @pl.when(s == 15)
            def _():
                pltpu.sync_copy(acc.at[pl.ds(9600, 400)],
                                tmp_out.at[t, pl.ds(9600, 400)])

    return _scatter


_scatter128 = _make_scatter_kernel(D_HID)
_scatter64 = _make_scatter_kernel(D_OUT)

BR = 1000


def _k2_body(x0, x1, x2, x3, w, deg, h1p):
    for t in range(NT):
        x = (x0, x1, x2, x3)[t]
        dinv = lax.rsqrt(deg[t] + 1.0)
        h1p[t] = dinv * jnp.dot(x[...], w[t], preferred_element_type=jnp.float32)


def _k4_body(tmp1, h1p, w2, b1, deg, h2p):
    for t in range(NT):
        dinv = lax.rsqrt(deg[t] + 1.0)
        h1 = _lrelu(dinv * (tmp1[t] + h1p[t]) + b1[t])
        h2p[t] = dinv * jnp.dot(h1, w2[t], preferred_element_type=jnp.float32)


def _k6_body(tmp2, h2p, b2, deg, xlj, xpj, wrl, brl, wrp, brp,
             comb_l, comb_p, c0, c1, c2, c3):
    conv = []
    for t in range(NT):
        dinv = lax.rsqrt(deg[t] + 1.0)
        conv.append(_lrelu(dinv * (tmp2[t] + h2p[t]) + b2[t]))
    c0[...] = conv[0]
    c1[...] = conv[1]
    c2[...] = conv[2]
    c3[...] = conv[3]
    comb_l[...] = 0.5 * (conv[0] + conv[2]) + (
        jnp.dot(xlj[...], wrl[...], preferred_element_type=jnp.float32) + brl[...])
    comb_p[...] = 0.5 * (conv[1] + conv[3]) + (
        jnp.dot(xpj[...], wrp[...], preferred_element_type=jnp.float32) + brp[...])


def _row_spec(D):
    return pl.BlockSpec((NT, BR, D), lambda i: (0, i, 0))


def _full_spec(shape):
    nd = len(shape)
    return pl.BlockSpec(shape, lambda i: (0,) * nd)


_x_spec = pl.BlockSpec((BR, D_IN), lambda i: (i, 0))
_deg_spec = pl.BlockSpec((NT, BR, 1), lambda i: (0, i, 0))


def kernel(x_lnc_jaccard, x_prot_jaccard, x_lnc_blast, x_prot_blast,
           edge_index_lnc_jaccard, edge_index_prot_jaccard,
           edge_index_lnc_blast, edge_index_prot_blast,
           W_j1_lnc, b_j1_lnc, W_j1_prot, b_j1_prot,
           W_j2_lnc, b_j2_lnc, W_j2_prot, b_j2_prot,
           W_b1_lnc, b_b1_lnc, W_b1_prot, b_b1_prot,
           W_b2_lnc, b_b2_lnc, W_b2_prot, b_b2_prot,
           W_res_lnc, b_res_lnc, W_res_prot, b_res_prot):
    eis = (edge_index_lnc_jaccard, edge_index_prot_jaccard,
           edge_index_lnc_blast, edge_index_prot_blast)
    pad = EP - E
    pad_src = jnp.arange(pad, dtype=jnp.int32) * 13 % N
    pad_dst = PAD_DST + jnp.arange(pad, dtype=jnp.int32) % (N_ACC - PAD_DST)
    toff = (jnp.arange(NT, dtype=jnp.int32) * N)[:, None]
    srcs = jnp.concatenate(
        [jnp.stack([ei[0] for ei in eis]),
         jnp.broadcast_to(pad_src, (NT, pad))], axis=1) + toff
    dsts = jnp.concatenate(
        [jnp.stack([ei[1] for ei in eis]),
         jnp.broadcast_to(pad_dst, (NT, pad))], axis=1)
    IDX = jnp.stack([srcs.reshape(NT, NS, NCHUNK, CH),
                     dsts.reshape(NT, NS, NCHUNK, CH)], axis=3)

    deg = _deg_kernel(IDX)[:, :N]
    deg_r = deg.reshape(NT, N, 1)

    W1 = jnp.stack([W_j1_lnc, W_j1_prot, W_b1_lnc, W_b1_prot])
    b1 = jnp.stack([b_j1_lnc, b_j1_prot, b_b1_lnc, b_b1_prot]).reshape(NT, 1, D_HID)
    W2 = jnp.stack([W_j2_lnc, W_j2_prot, W_b2_lnc, W_b2_prot])
    b2 = jnp.stack([b_j2_lnc, b_j2_prot, b_b2_lnc, b_b2_prot]).reshape(NT, 1, D_OUT)

    H1 = pl.pallas_call(
        _k2_body,
        grid=(N // BR,),
        in_specs=[_x_spec, _x_spec, _x_spec, _x_spec,
                  _full_spec((NT, D_IN, D_HID)), _deg_spec],
        out_specs=_row_spec(D_HID),
        out_shape=jax.ShapeDtypeStruct((NT, N, D_HID), jnp.float32),
    )(x_lnc_jaccard, x_prot_jaccard, x_lnc_blast, x_prot_blast, W1, deg_r)

    TMP1 = _scatter128(IDX, H1.reshape(NT * N, D_HID))

    H2 = pl.pallas_call(
        _k4_body,
        grid=(N // BR,),
        in_specs=[_row_spec(D_HID), _row_spec(D_HID),
                  _full_spec((NT, D_HID, D_OUT)), _full_spec((NT, 1, D_HID)),
                  _deg_spec],
        out_specs=_row_spec(D_OUT),
        out_shape=jax.ShapeDtypeStruct((NT, N, D_OUT), jnp.float32),
    )(TMP1, H1, W2, b1, deg_r)

    TMP2 = _scatter64(IDX, H2.reshape(NT * N, D_OUT))

    o_spec = pl.BlockSpec((BR, D_OUT), lambda i: (i, 0))
    o_shape = jax.ShapeDtypeStruct((N, D_OUT), jnp.float32)
    comb_l, comb_p, c0, c1, c2, c3 = pl.pallas_call(
        _k6_body,
        grid=(N // BR,),
        in_specs=[_row_spec(D_OUT), _row_spec(D_OUT), _full_spec((NT, 1, D_OUT)),
                  _deg_spec, _x_spec, _x_spec,
                  _full_spec((D_IN, D_OUT)), _full_spec((1, D_OUT)),
                  _full_spec((D_IN, D_OUT)), _full_spec((1, D_OUT))],
        out_specs=[o_spec] * 6,
        out_shape=[o_shape] * 6,
    )(TMP2, H2, b2, deg_r, x_lnc_jaccard, x_prot_jaccard,
      W_res_lnc, b_res_lnc.reshape(1, D_OUT), W_res_prot,
      b_res_prot.reshape(1, D_OUT))

    return (comb_l, comb_p, c0, c1, c2, c3)


# speedup vs baseline: 1.0045x; 1.0045x over previous
import functools

import jax
import jax.numpy as jnp
from jax import lax
from jax.experimental import pallas as pl
from jax.experimental.pallas import tpu as pltpu
from jax.experimental.pallas import tpu_sc as plsc

N = 10000
E = 320000
NT = 4
D_IN = 128
D_HID = 128
D_OUT = 64

NC = 2
NS = 16
CH = 128
G = 4
NCHUNK = 160
NG = NCHUNK // G
E_PT = NCHUNK * CH
EP = E_PT * NS
N_ACC = 10240
ROWS_PT = 640
PAD_DST = 10016

_mesh = plsc.VectorSubcoreMesh(core_axis_name="c", subcore_axis_name="s")
_sc_params = pltpu.CompilerParams(use_tc_tiling_on_sc=False)


def _lrelu(x):
    return jnp.where(x >= 0, x, 0.2 * x)


@functools.partial(
    pl.kernel,
    out_type=jax.ShapeDtypeStruct((NT, N_ACC), jnp.float32),
    mesh=_mesh,
    compiler_params=_sc_params,
    scratch_types=[
        pltpu.VMEM((2, CH), jnp.int32),
        pltpu.VMEM((2, CH), jnp.int32),
        pltpu.VMEM((2, CH), jnp.int32),
        pltpu.VMEM((2, CH), jnp.int32),
        pltpu.VMEM((CH,), jnp.float32),
        pltpu.VMEM((ROWS_PT,), jnp.float32),
        pltpu.VMEM_SHARED((N_ACC,), jnp.float32),
        pltpu.SemaphoreType.DMA,
        pltpu.SemaphoreType.DMA,
        pltpu.SemaphoreType.DMA,
        pltpu.SemaphoreType.DMA,
        pltpu.SemaphoreType.DMA,
        pltpu.SemaphoreType.DMA,
    ],
)
def _deg_kernel(idx_hbm, deg_out, ib0, ib1, ib2, ib3, ones_v, zeros_v, acc,
                isem0, isem1, isem2, isem3, ssem0, ssem1):
    c = lax.axis_index("c")
    s = lax.axis_index("s")
    ib = (ib0, ib1, ib2, ib3)
    isem = (isem0, isem1, isem2, isem3)
    ssem = (ssem0, ssem1)
    for i in range(8):
        ones_v[pl.ds(16 * i, 16)] = jnp.ones((16,), jnp.float32)
    for i in range(ROWS_PT // 16):
        zeros_v[pl.ds(16 * i, 16)] = jnp.zeros((16,), jnp.float32)
    for k in range(2):
        t = 2 * c + k
        pltpu.sync_copy(zeros_v, acc.at[pl.ds(s * ROWS_PT, ROWS_PT)])
        plsc.subcore_barrier()

        pltpu.sync_copy(idx_hbm.at[t, 0, s], ib0)
        pltpu.async_copy(idx_hbm.at[t, 1, s], ib1, isem1)

        def quad(q, _):
            for u in range(4):
                j = 4 * q + u
                I0, I2 = u % 4, (u + 2) % 4
                X = u % 2
                Y = 1 - X
                if u == 0:
                    @pl.when(j > 0)
                    def _():
                        pltpu.make_async_copy(ones_v, acc.at[ib[I0].at[1]],
                                              ssem[Y]).wait()
                else:
                    pltpu.make_async_copy(ones_v, acc.at[ib[I0].at[1]],
                                          ssem[Y]).wait()
                jp2 = jnp.minimum(j + 2, NCHUNK - 1)
                pltpu.async_copy(idx_hbm.at[t, jp2, s], ib[I2],
                                 isem[I2])
                if u == 0:
                    @pl.when(j > 0)
                    def _():
                        pltpu.make_async_copy(idx_hbm.at[t, 0, s],
                                              ib[I0], isem[I0]).wait()
                else:
                    pltpu.make_async_copy(idx_hbm.at[t, 0, s],
                                          ib[I0], isem[I0]).wait()
                pltpu.async_copy(ones_v, acc.at[ib[I0].at[1]], ssem[X],
                                 add=True)
            return 0
        lax.fori_loop(0, NCHUNK // 4, quad, 0)
        pltpu.make_async_copy(ones_v, acc.at[ib3.at[1]], ssem1).wait()
        pltpu.make_async_copy(idx_hbm.at[t, 0, s], ib0, isem0).wait()
        pltpu.make_async_copy(idx_hbm.at[t, 0, s], ib1, isem1).wait()
        plsc.subcore_barrier()

        pltpu.sync_copy(acc.at[pl.ds(s * ROWS_PT, ROWS_PT)],
                        deg_out.at[t, pl.ds(s * ROWS_PT, ROWS_PT)])


def _make_scatter_kernel(D):
    @functools.partial(
        pl.kernel,
        out_type=jax.ShapeDtypeStruct((NT, N, D), jnp.float32),
        mesh=_mesh,
        compiler_params=_sc_params,
        scratch_types=[
            pltpu.VMEM((2, CH), jnp.int32),
            pltpu.VMEM((2, CH), jnp.int32),
            pltpu.VMEM((2, CH), jnp.int32),
            pltpu.VMEM((2, CH), jnp.int32),
            pltpu.VMEM((CH, D), jnp.float32),
            pltpu.VMEM((CH, D), jnp.float32),
            pltpu.VMEM((64, D), jnp.float32),
            pltpu.VMEM_SHARED((N_ACC, D), jnp.float32),
            pltpu.SemaphoreType.DMA,
            pltpu.SemaphoreType.DMA,
            pltpu.SemaphoreType.DMA,
            pltpu.SemaphoreType.DMA,
            pltpu.SemaphoreType.DMA,
            pltpu.SemaphoreType.DMA,
            pltpu.SemaphoreType.DMA,
            pltpu.SemaphoreType.DMA,
        ],
    )
    def _scatter(idx_hbm, h_hbm, tmp_out, ib0, ib1, ib2, ib3, rows0, rows1,
                 zbig, acc, isem0, isem1, isem2, isem3,
                 gsem0, gsem1, ssem0, ssem1):
        c = lax.axis_index("c")
        s = lax.axis_index("s")
        ib = (ib0, ib1, ib2, ib3)
        isem = (isem0, isem1, isem2, isem3)
        rowsb = (rows0, rows1)
        gsem = (gsem0, gsem1)
        ssem = (ssem0, ssem1)
        def zfill(r, _):
            for q in range(D // 16):
                zbig[r, pl.ds(16 * q, 16)] = jnp.zeros((16,), jnp.float32)
            return 0
        lax.fori_loop(0, 64, zfill, 0)
        for k in range(2):
            t = 2 * c + k

            def zrow(j, _):
                pltpu.sync_copy(zbig, acc.at[pl.ds(s * ROWS_PT + 64 * j, 64)])
                return 0
            lax.fori_loop(0, ROWS_PT // 64, zrow, 0)
            plsc.subcore_barrier()

            pltpu.sync_copy(idx_hbm.at[t, 0, s], ib0)
            pltpu.async_copy(idx_hbm.at[t, 1, s], ib1, isem1)
            pltpu.async_copy(h_hbm.at[ib0.at[0]], rows0, gsem0)

            def quad(q, _):
                for u in range(4):
                    j = 4 * q + u
                    I0, I1, I2 = u % 4, (u + 1) % 4, (u + 2) % 4
                    X = u % 2
                    Y = 1 - X
                    if u == 0:
                        @pl.when(j > 0)
                        def _():
                            pltpu.make_async_copy(
                                rowsb[Y], acc.at[ib[I1].at[1]], ssem[Y]).wait()
                    else:
                        pltpu.make_async_copy(
                            rowsb[Y], acc.at[ib[I1].at[1]], ssem[Y]).wait()
                    jp2 = jnp.minimum(j + 2, NCHUNK - 1)
                    pltpu.async_copy(idx_hbm.at[t, jp2, s], ib[I2],
                                     isem[I2])
                    pltpu.make_async_copy(idx_hbm.at[t, 0, s], ib[I1],
                                          isem[I1]).wait()
                    pltpu.async_copy(h_hbm.at[ib[I1].at[0]], rowsb[Y],
                                     gsem[Y])
                    pltpu.make_async_copy(h_hbm.at[ib[I0].at[0]], rowsb[X],
                                          gsem[X]).wait()
                    pltpu.async_copy(rowsb[X], acc.at[ib[I0].at[1]],
                                     ssem[X], add=True)
                return 0
            lax.fori_loop(0, NCHUNK // 4, quad, 0)
            pltpu.make_async_copy(h_hbm.at[ib0.at[0]], rows0, gsem0).wait()
            pltpu.make_async_copy(rows1, acc.at[ib3.at[1]], ssem1).wait()
            pltpu.make_async_copy(idx_hbm.at[t, 0, s], ib1,
                                  isem1).wait()
            plsc.subcore_barrier()

            @pl.when(s < 15)
            def _():
                pltpu.sync_copy(acc.at[pl.ds(s * ROWS_PT, ROWS_PT)],
                                tmp_out.at[t, pl.ds(s * ROWS_PT, ROWS_PT)])

            @pl.when(s == 15)
            def _():
                pltpu.sync_copy(acc.at[pl.ds(9600, 400)],
                                tmp_out.at[t, pl.ds(9600, 400)])

    return _scatter


_scatter128 = _make_scatter_kernel(D_HID)
_scatter64 = _make_scatter_kernel(D_OUT)

BR = 1000


def _k2_body(x0, x1, x2, x3, w, deg, h1p):
    for t in range(NT):
        x = (x0, x1, x2, x3)[t]
        dinv = lax.rsqrt(deg[t] + 1.0)
        h1p[t] = dinv * jnp.dot(x[...], w[t], preferred_element_type=jnp.float32)


def _k4_body(tmp1, h1p, w2, b1, deg, h2p):
    for t in range(NT):
        dinv = lax.rsqrt(deg[t] + 1.0)
        h1 = _lrelu(dinv * (tmp1[t] + h1p[t]) + b1[t])
        h2p[t] = dinv * jnp.dot(h1, w2[t], preferred_element_type=jnp.float32)


def _k6_body(tmp2, h2p, b2, deg, xlj, xpj, wrl, brl, wrp, brp,
             comb_l, comb_p, c0, c1, c2, c3):
    conv = []
    for t in range(NT):
        dinv = lax.rsqrt(deg[t] + 1.0)
        conv.append(_lrelu(dinv * (tmp2[t] + h2p[t]) + b2[t]))
    c0[...] = conv[0]
    c1[...] = conv[1]
    c2[...] = conv[2]
    c3[...] = conv[3]
    comb_l[...] = 0.5 * (conv[0] + conv[2]) + (
        jnp.dot(xlj[...], wrl[...], preferred_element_type=jnp.float32) + brl[...])
    comb_p[...] = 0.5 * (conv[1] + conv[3]) + (
        jnp.dot(xpj[...], wrp[...], preferred_element_type=jnp.float32) + brp[...])


def _row_spec(D):
    return pl.BlockSpec((NT, BR, D), lambda i: (0, i, 0))


def _full_spec(shape):
    nd = len(shape)
    return pl.BlockSpec(shape, lambda i: (0,) * nd)


_x_spec = pl.BlockSpec((BR, D_IN), lambda i: (i, 0))
_deg_spec = pl.BlockSpec((NT, BR, 1), lambda i: (0, i, 0))


def kernel(x_lnc_jaccard, x_prot_jaccard, x_lnc_blast, x_prot_blast,
           edge_index_lnc_jaccard, edge_index_prot_jaccard,
           edge_index_lnc_blast, edge_index_prot_blast,
           W_j1_lnc, b_j1_lnc, W_j1_prot, b_j1_prot,
           W_j2_lnc, b_j2_lnc, W_j2_prot, b_j2_prot,
           W_b1_lnc, b_b1_lnc, W_b1_prot, b_b1_prot,
           W_b2_lnc, b_b2_lnc, W_b2_prot, b_b2_prot,
           W_res_lnc, b_res_lnc, W_res_prot, b_res_prot):
    eis = (edge_index_lnc_jaccard, edge_index_prot_jaccard,
           edge_index_lnc_blast, edge_index_prot_blast)
    pad = EP - E
    pad_src = jnp.arange(pad, dtype=jnp.int32) * 13 % N
    pad_dst = PAD_DST + jnp.arange(pad, dtype=jnp.int32) % (N_ACC - PAD_DST)
    toff = (jnp.arange(NT, dtype=jnp.int32) * N)[:, None]
    srcs = jnp.concatenate(
        [jnp.stack([ei[0] for ei in eis]),
         jnp.broadcast_to(pad_src, (NT, pad))], axis=1) + toff
    dsts = jnp.concatenate(
        [jnp.stack([ei[1] for ei in eis]),
         jnp.broadcast_to(pad_dst, (NT, pad))], axis=1)
    IDX = jnp.stack([srcs.reshape(NT, NS, NCHUNK, CH),
                     dsts.reshape(NT, NS, NCHUNK, CH)],
                    axis=3).transpose(0, 2, 1, 3, 4)

    deg = _deg_kernel(IDX)[:, :N]
    deg_r = deg.reshape(NT, N, 1)

    W1 = jnp.stack([W_j1_lnc, W_j1_prot, W_b1_lnc, W_b1_prot])
    b1 = jnp.stack([b_j1_lnc, b_j1_prot, b_b1_lnc, b_b1_prot]).reshape(NT, 1, D_HID)
    W2 = jnp.stack([W_j2_lnc, W_j2_prot, W_b2_lnc, W_b2_prot])
    b2 = jnp.stack([b_j2_lnc, b_j2_prot, b_b2_lnc, b_b2_prot]).reshape(NT, 1, D_OUT)

    H1 = pl.pallas_call(
        _k2_body,
        grid=(N // BR,),
        in_specs=[_x_spec, _x_spec, _x_spec, _x_spec,
                  _full_spec((NT, D_IN, D_HID)), _deg_spec],
        out_specs=_row_spec(D_HID),
        out_shape=jax.ShapeDtypeStruct((NT, N, D_HID), jnp.float32),
    )(x_lnc_jaccard, x_prot_jaccard, x_lnc_blast, x_prot_blast, W1, deg_r)

    TMP1 = _scatter128(IDX, H1.reshape(NT * N, D_HID))

    H2 = pl.pallas_call(
        _k4_body,
        grid=(N // BR,),
        in_specs=[_row_spec(D_HID), _row_spec(D_HID),
                  _full_spec((NT, D_HID, D_OUT)), _full_spec((NT, 1, D_HID)),
                  _deg_spec],
        out_specs=_row_spec(D_OUT),
        out_shape=jax.ShapeDtypeStruct((NT, N, D_OUT), jnp.float32),
    )(TMP1, H1, W2, b1, deg_r)

    TMP2 = _scatter64(IDX, H2.reshape(NT * N, D_OUT))

    o_spec = pl.BlockSpec((BR, D_OUT), lambda i: (i, 0))
    o_shape = jax.ShapeDtypeStruct((N, D_OUT), jnp.float32)
    comb_l, comb_p, c0, c1, c2, c3 = pl.pallas_call(
        _k6_body,
        grid=(N // BR,),
        in_specs=[_row_spec(D_OUT), _row_spec(D_OUT), _full_spec((NT, 1, D_OUT)),
                  _deg_spec, _x_spec, _x_spec,
                  _full_spec((D_IN, D_OUT)), _full_spec((1, D_OUT)),
                  _full_spec((D_IN, D_OUT)), _full_spec((1, D_OUT))],
        out_specs=[o_spec] * 6,
        out_shape=[o_shape] * 6,
    )(TMP2, H2, b2, deg_r, x_lnc_jaccard, x_prot_jaccard,
      W_res_lnc, b_res_lnc.reshape(1, D_OUT), W_res_prot,
      b_res_prot.reshape(1, D_OUT))

    return (comb_l, comb_p, c0, c1, c2, c3)


# speedup vs baseline: 1.0594x; 1.0546x over previous
import functools

import jax
import jax.numpy as jnp
from jax import lax
from jax.experimental import pallas as pl
from jax.experimental.pallas import tpu as pltpu
from jax.experimental.pallas import tpu_sc as plsc

N = 10000
E = 320000
NT = 4
D_IN = 128
D_HID = 128
D_OUT = 64

NC = 2
NS = 16
CH = 128
NCHUNK = 160
E_PT = NCHUNK * CH
EP = E_PT * NS
N_ACC = 10240
ROWS_PT = 640
PAD_DST = 10016

_mesh = plsc.VectorSubcoreMesh(core_axis_name="c", subcore_axis_name="s")
_sc_params = pltpu.CompilerParams(use_tc_tiling_on_sc=False)


def _lrelu(x):
    return jnp.where(x >= 0, x, 0.2 * x)


def _make_deg_kernel(p):
    @functools.partial(
        pl.kernel,
        out_type=jax.ShapeDtypeStruct((2, N_ACC), jnp.float32),
        mesh=_mesh,
        compiler_params=_sc_params,
        scratch_types=[
            pltpu.VMEM((2, CH), jnp.int32),
            pltpu.VMEM((2, CH), jnp.int32),
            pltpu.VMEM((2, CH), jnp.int32),
            pltpu.VMEM((2, CH), jnp.int32),
            pltpu.VMEM((CH,), jnp.float32),
            pltpu.VMEM((ROWS_PT,), jnp.float32),
            pltpu.VMEM_SHARED((N_ACC,), jnp.float32),
            pltpu.SemaphoreType.DMA,
            pltpu.SemaphoreType.DMA,
            pltpu.SemaphoreType.DMA,
            pltpu.SemaphoreType.DMA,
            pltpu.SemaphoreType.DMA,
            pltpu.SemaphoreType.DMA,
        ],
    )
    def _deg(idx_hbm, deg_out, ib0, ib1, ib2, ib3, ones_v, zeros_v, acc,
             isem0, isem1, isem2, isem3, ssem0, ssem1):
        c = lax.axis_index("c")
        s = lax.axis_index("s")
        t = 2 * p + c
        ib = (ib0, ib1, ib2, ib3)
        isem = (isem0, isem1, isem2, isem3)
        ssem = (ssem0, ssem1)
        for i in range(8):
            ones_v[pl.ds(16 * i, 16)] = jnp.ones((16,), jnp.float32)
        for i in range(ROWS_PT // 16):
            zeros_v[pl.ds(16 * i, 16)] = jnp.zeros((16,), jnp.float32)
        pltpu.sync_copy(zeros_v, acc.at[pl.ds(s * ROWS_PT, ROWS_PT)])
        plsc.subcore_barrier()

        pltpu.sync_copy(idx_hbm.at[t, 0, s], ib0)
        pltpu.async_copy(idx_hbm.at[t, 1, s], ib1, isem1)

        def quad(q, _):
            for u in range(4):
                j = 4 * q + u
                I0, I2 = u % 4, (u + 2) % 4
                X = u % 2
                Y = 1 - X
                if u == 0:
                    @pl.when(j > 0)
                    def _():
                        pltpu.make_async_copy(ones_v, acc.at[ib[I0].at[1]],
                                              ssem[Y]).wait()
                else:
                    pltpu.make_async_copy(ones_v, acc.at[ib[I0].at[1]],
                                          ssem[Y]).wait()
                jp2 = jnp.minimum(j + 2, NCHUNK - 1)
                pltpu.async_copy(idx_hbm.at[t, jp2, s], ib[I2], isem[I2])
                if u == 0:
                    @pl.when(j > 0)
                    def _():
                        pltpu.make_async_copy(idx_hbm.at[t, 0, s],
                                              ib[I0], isem[I0]).wait()
                else:
                    pltpu.make_async_copy(idx_hbm.at[t, 0, s],
                                          ib[I0], isem[I0]).wait()
                pltpu.async_copy(ones_v, acc.at[ib[I0].at[1]], ssem[X],
                                 add=True)
            return 0
        lax.fori_loop(0, NCHUNK // 4, quad, 0)
        pltpu.make_async_copy(ones_v, acc.at[ib3.at[1]], ssem1).wait()
        pltpu.make_async_copy(idx_hbm.at[t, 0, s], ib0, isem0).wait()
        pltpu.make_async_copy(idx_hbm.at[t, 0, s], ib1, isem1).wait()
        plsc.subcore_barrier()

        pltpu.sync_copy(acc.at[pl.ds(s * ROWS_PT, ROWS_PT)],
                        deg_out.at[c, pl.ds(s * ROWS_PT, ROWS_PT)])

    return _deg


def _make_scatter_kernel(D, p):
    @functools.partial(
        pl.kernel,
        out_type=jax.ShapeDtypeStruct((2, N, D), jnp.float32),
        mesh=_mesh,
        compiler_params=_sc_params,
        scratch_types=[
            pltpu.VMEM((2, CH), jnp.int32),
            pltpu.VMEM((2, CH), jnp.int32),
            pltpu.VMEM((2, CH), jnp.int32),
            pltpu.VMEM((2, CH), jnp.int32),
            pltpu.VMEM((CH, D), jnp.float32),
            pltpu.VMEM((CH, D), jnp.float32),
            pltpu.VMEM((64, D), jnp.float32),
            pltpu.VMEM_SHARED((N_ACC, D), jnp.float32),
            pltpu.SemaphoreType.DMA,
            pltpu.SemaphoreType.DMA,
            pltpu.SemaphoreType.DMA,
            pltpu.SemaphoreType.DMA,
            pltpu.SemaphoreType.DMA,
            pltpu.SemaphoreType.DMA,
            pltpu.SemaphoreType.DMA,
            pltpu.SemaphoreType.DMA,
        ],
    )
    def _scatter(idx_hbm, h_hbm, tmp_out, ib0, ib1, ib2, ib3, rows0, rows1,
                 zbig, acc, isem0, isem1, isem2, isem3,
                 gsem0, gsem1, ssem0, ssem1):
        c = lax.axis_index("c")
        s = lax.axis_index("s")
        t = 2 * p + c
        ib = (ib0, ib1, ib2, ib3)
        isem = (isem0, isem1, isem2, isem3)
        rowsb = (rows0, rows1)
        gsem = (gsem0, gsem1)
        ssem = (ssem0, ssem1)

        def zfill(r, _):
            for q in range(D // 16):
                zbig[r, pl.ds(16 * q, 16)] = jnp.zeros((16,), jnp.float32)
            return 0
        lax.fori_loop(0, 64, zfill, 0)

        def zrow(j, _):
            pltpu.sync_copy(zbig, acc.at[pl.ds(s * ROWS_PT + 64 * j, 64)])
            return 0
        lax.fori_loop(0, ROWS_PT // 64, zrow, 0)
        plsc.subcore_barrier()

        pltpu.sync_copy(idx_hbm.at[t, 0, s], ib0)
        pltpu.async_copy(idx_hbm.at[t, 1, s], ib1, isem1)
        pltpu.async_copy(h_hbm.at[ib0.at[0]], rows0, gsem0)

        def quad(q, _):
            for u in range(4):
                j = 4 * q + u
                I0, I1, I2 = u % 4, (u + 1) % 4, (u + 2) % 4
                X = u % 2
                Y = 1 - X
                if u == 0:
                    @pl.when(j > 0)
                    def _():
                        pltpu.make_async_copy(
                            rowsb[Y], acc.at[ib[I1].at[1]], ssem[Y]).wait()
                else:
                    pltpu.make_async_copy(
                        rowsb[Y], acc.at[ib[I1].at[1]], ssem[Y]).wait()
                jp2 = jnp.minimum(j + 2, NCHUNK - 1)
                pltpu.async_copy(idx_hbm.at[t, jp2, s], ib[I2], isem[I2])
                pltpu.make_async_copy(idx_hbm.at[t, 0, s], ib[I1],
                                      isem[I1]).wait()
                pltpu.async_copy(h_hbm.at[ib[I1].at[0]], rowsb[Y], gsem[Y])
                pltpu.make_async_copy(h_hbm.at[ib[I0].at[0]], rowsb[X],
                                      gsem[X]).wait()
                pltpu.async_copy(rowsb[X], acc.at[ib[I0].at[1]],
                                 ssem[X], add=True)
            return 0
        lax.fori_loop(0, NCHUNK // 4, quad, 0)
        pltpu.make_async_copy(h_hbm.at[ib0.at[0]], rows0, gsem0).wait()
        pltpu.make_async_copy(rows1, acc.at[ib3.at[1]], ssem1).wait()
        pltpu.make_async_copy(idx_hbm.at[t, 0, s], ib1, isem1).wait()
        plsc.subcore_barrier()

        @pl.when(s < 15)
        def _():
            pltpu.sync_copy(acc.at[pl.ds(s * ROWS_PT, ROWS_PT)],
                            tmp_out.at[c, pl.ds(s * ROWS_PT, ROWS_PT)])

        @pl.when(s == 15)
        def _():
            pltpu.sync_copy(acc.at[pl.ds(9600, 400)],
                            tmp_out.at[c, pl.ds(9600, 400)])

    return _scatter


_deg_p = tuple(_make_deg_kernel(p) for p in range(2))
_scat128_p = tuple(_make_scatter_kernel(D_HID, p) for p in range(2))
_scat64_p = tuple(_make_scatter_kernel(D_OUT, p) for p in range(2))

BR = 1000


def _k2_body(x0, x1, w, deg, h1p):
    for u in range(2):
        x = (x0, x1)[u]
        dinv = lax.rsqrt(deg[u] + 1.0)
        h1p[u] = dinv * jnp.dot(x[...], w[u], preferred_element_type=jnp.float32)


def _k4_body(tmp1, h1p, w2, b1, deg, h2p):
    for u in range(2):
        dinv = lax.rsqrt(deg[u] + 1.0)
        h1 = _lrelu(dinv * (tmp1[u] + h1p[u]) + b1[u])
        h2p[u] = dinv * jnp.dot(h1, w2[u], preferred_element_type=jnp.float32)


def _k6_body(tmp2a, tmp2b, h2pa, h2pb, b2, dega, degb, xlj, xpj,
             wrl, brl, wrp, brp, comb_l, comb_p, c0, c1, c2, c3):
    conv = []
    for t in range(NT):
        p, u = divmod(t, 2)
        tmp2 = (tmp2a, tmp2b)[p]
        h2p = (h2pa, h2pb)[p]
        deg = (dega, degb)[p]
        dinv = lax.rsqrt(deg[u] + 1.0)
        conv.append(_lrelu(dinv * (tmp2[u] + h2p[u]) + b2[t]))
    c0[...] = conv[0]
    c1[...] = conv[1]
    c2[...] = conv[2]
    c3[...] = conv[3]
    comb_l[...] = 0.5 * (conv[0] + conv[2]) + (
        jnp.dot(xlj[...], wrl[...], preferred_element_type=jnp.float32) + brl[...])
    comb_p[...] = 0.5 * (conv[1] + conv[3]) + (
        jnp.dot(xpj[...], wrp[...], preferred_element_type=jnp.float32) + brp[...])


def _row_spec(D):
    return pl.BlockSpec((2, BR, D), lambda i: (0, i, 0))


def _full_spec(shape):
    nd = len(shape)
    return pl.BlockSpec(shape, lambda i: (0,) * nd)


_x_spec = pl.BlockSpec((BR, D_IN), lambda i: (i, 0))
_deg_spec = pl.BlockSpec((2, BR, 1), lambda i: (0, i, 0))


def kernel(x_lnc_jaccard, x_prot_jaccard, x_lnc_blast, x_prot_blast,
           edge_index_lnc_jaccard, edge_index_prot_jaccard,
           edge_index_lnc_blast, edge_index_prot_blast,
           W_j1_lnc, b_j1_lnc, W_j1_prot, b_j1_prot,
           W_j2_lnc, b_j2_lnc, W_j2_prot, b_j2_prot,
           W_b1_lnc, b_b1_lnc, W_b1_prot, b_b1_prot,
           W_b2_lnc, b_b2_lnc, W_b2_prot, b_b2_prot,
           W_res_lnc, b_res_lnc, W_res_prot, b_res_prot):
    eis = (edge_index_lnc_jaccard, edge_index_prot_jaccard,
           edge_index_lnc_blast, edge_index_prot_blast)
    xs = (x_lnc_jaccard, x_prot_jaccard, x_lnc_blast, x_prot_blast)
    pad = EP - E
    pad_src = jnp.arange(pad, dtype=jnp.int32) * 13 % N
    pad_dst = PAD_DST + jnp.arange(pad, dtype=jnp.int32) % (N_ACC - PAD_DST)
    toff = ((jnp.arange(NT, dtype=jnp.int32) % 2) * N)[:, None]
    srcs = jnp.concatenate(
        [jnp.stack([ei[0] for ei in eis]),
         jnp.broadcast_to(pad_src, (NT, pad))], axis=1) + toff
    dsts = jnp.concatenate(
        [jnp.stack([ei[1] for ei in eis]),
         jnp.broadcast_to(pad_dst, (NT, pad))], axis=1)
    IDX = jnp.stack([srcs.reshape(NT, NS, NCHUNK, CH),
                     dsts.reshape(NT, NS, NCHUNK, CH)],
                    axis=3).transpose(0, 2, 1, 3, 4)

    W1 = (jnp.stack([W_j1_lnc, W_j1_prot]), jnp.stack([W_b1_lnc, W_b1_prot]))
    b1 = (jnp.stack([b_j1_lnc, b_j1_prot]).reshape(2, 1, D_HID),
          jnp.stack([b_b1_lnc, b_b1_prot]).reshape(2, 1, D_HID))
    W2 = (jnp.stack([W_j2_lnc, W_j2_prot]), jnp.stack([W_b2_lnc, W_b2_prot]))
    b2 = jnp.stack([b_j2_lnc, b_j2_prot, b_b2_lnc, b_b2_prot]).reshape(
        NT, 1, D_OUT)

    degs, h2s, tmp2s = [], [], []
    for p in range(2):
        deg = _deg_p[p](IDX)
        deg_r = deg[:, :N].reshape(2, N, 1)
        H1 = pl.pallas_call(
            _k2_body,
            grid=(N // BR,),
            in_specs=[_x_spec, _x_spec, _full_spec((2, D_IN, D_HID)),
                      _deg_spec],
            out_specs=_row_spec(D_HID),
            out_shape=jax.ShapeDtypeStruct((2, N, D_HID), jnp.float32),
        )(xs[2 * p], xs[2 * p + 1], W1[p], deg_r)

        TMP1 = _scat128_p[p](IDX, H1.reshape(2 * N, D_HID))

        H2 = pl.pallas_call(
            _k4_body,
            grid=(N // BR,),
            in_specs=[_row_spec(D_HID), _row_spec(D_HID),
                      _full_spec((2, D_HID, D_OUT)),
                      _full_spec((2, 1, D_HID)), _deg_spec],
            out_specs=_row_spec(D_OUT),
            out_shape=jax.ShapeDtypeStruct((2, N, D_OUT), jnp.float32),
        )(TMP1, H1, W2[p], b1[p], deg_r)

        TMP2 = _scat64_p[p](IDX, H2.reshape(2 * N, D_OUT))
        degs.append(deg_r)
        h2s.append(H2)
        tmp2s.append(TMP2)

    o_spec = pl.BlockSpec((BR, D_OUT), lambda i: (i, 0))
    o_shape = jax.ShapeDtypeStruct((N, D_OUT), jnp.float32)
    comb_l, comb_p, c0, c1, c2, c3 = pl.pallas_call(
        _k6_body,
        grid=(N // BR,),
        in_specs=[_row_spec(D_OUT), _row_spec(D_OUT),
                  _row_spec(D_OUT), _row_spec(D_OUT),
                  _full_spec((NT, 1, D_OUT)), _deg_spec, _deg_spec,
                  _x_spec, _x_spec,
                  _full_spec((D_IN, D_OUT)), _full_spec((1, D_OUT)),
                  _full_spec((D_IN, D_OUT)), _full_spec((1, D_OUT))],
        out_specs=[o_spec] * 6,
        out_shape=[o_shape] * 6,
    )(tmp2s[0], tmp2s[1], h2s[0], h2s[1], b2, degs[0], degs[1],
      x_lnc_jaccard, x_prot_jaccard,
      W_res_lnc, b_res_lnc.reshape(1, D_OUT), W_res_prot,
      b_res_prot.reshape(1, D_OUT))

    return (comb_l, comb_p, c0, c1, c2, c3)
